# baseline (speedup 1.0000x reference)
import jax
import jax.numpy as jnp
from jax import lax
from jax.experimental import pallas as pl
from jax.experimental.pallas import tpu as pltpu
from jax.experimental.pallas import tpu_sc as plsc

BATCH = 16384
EMB_DIM = 32
NUM_WORKERS = 32
B_PER_W = BATCH // NUM_WORKERS
CHUNK = 128
N_CHUNKS = B_PER_W // CHUNK
LANES = 16


def _simple_body(heads_hbm, rels_hbm, tails_hbm, eh_hbm, et_hbm, rf_hbm, ri_hbm,
                 out_hbm,
                 h_idx, r_idx, t_idx,
                 h1, t1, h2, t2, r1, r2,
                 out_v, sem):
  wid = lax.axis_index("s") * 2 + lax.axis_index("c")
  base_tile = wid * N_CHUNKS

  pltpu.sync_copy(heads_hbm.at[pl.ds(base_tile, N_CHUNKS)], h_idx)
  pltpu.sync_copy(rels_hbm.at[pl.ds(base_tile, N_CHUNKS)], r_idx)
  pltpu.sync_copy(tails_hbm.at[pl.ds(base_tile, N_CHUNKS)], t_idx)

  copies = []
  for j in range(N_CHUNKS):
    rows = pl.ds(j * CHUNK, CHUNK)
    copies.append(pltpu.async_copy(eh_hbm.at[h_idx.at[j]], h1.at[rows], sem))
    copies.append(pltpu.async_copy(et_hbm.at[t_idx.at[j]], t1.at[rows], sem))
    copies.append(pltpu.async_copy(et_hbm.at[h_idx.at[j]], h2.at[rows], sem))
    copies.append(pltpu.async_copy(eh_hbm.at[t_idx.at[j]], t2.at[rows], sem))
    copies.append(pltpu.async_copy(rf_hbm.at[r_idx.at[j]], r1.at[rows], sem))
    copies.append(pltpu.async_copy(ri_hbm.at[r_idx.at[j]], r2.at[rows], sem))
  for c in copies:
    c.wait()

  lane = lax.iota(jnp.int32, LANES)
  lo = pl.ds(0, LANES)
  hi = pl.ds(LANES, LANES)

  def group(i, carry):
    acc = jnp.zeros((LANES,), jnp.float32)
    for k in range(LANES):
      row = i * LANES + k
      a = (h1[row, lo] * r1[row, lo] * t1[row, lo]
           + h2[row, lo] * r2[row, lo] * t2[row, lo])
      b = (h1[row, hi] * r1[row, hi] * t1[row, hi]
           + h2[row, hi] * r2[row, hi] * t2[row, hi])
      acc = jnp.where(lane == k, jnp.sum(a + b), acc)
    out_v[pl.ds(i * LANES, LANES)] = acc * 0.5
    return carry

  lax.fori_loop(0, B_PER_W // LANES, group, 0)

  pltpu.sync_copy(out_v, out_hbm.at[pl.ds(wid * B_PER_W, B_PER_W)])


@jax.jit
def _simple_sc(heads, rels, tails, eh, et, rf, ri):
  mesh = plsc.VectorSubcoreMesh(core_axis_name="c", subcore_axis_name="s")
  run = pl.kernel(
      _simple_body,
      out_type=jax.ShapeDtypeStruct((BATCH,), jnp.float32),
      mesh=mesh,
      compiler_params=pltpu.CompilerParams(
          needs_layout_passes=False, use_tc_tiling_on_sc=False),
      scratch_types=[
          pltpu.VMEM((N_CHUNKS, CHUNK), jnp.int32),
          pltpu.VMEM((N_CHUNKS, CHUNK), jnp.int32),
          pltpu.VMEM((N_CHUNKS, CHUNK), jnp.int32),
          pltpu.VMEM((B_PER_W, EMB_DIM), jnp.float32),
          pltpu.VMEM((B_PER_W, EMB_DIM), jnp.float32),
          pltpu.VMEM((B_PER_W, EMB_DIM), jnp.float32),
          pltpu.VMEM((B_PER_W, EMB_DIM), jnp.float32),
          pltpu.VMEM((B_PER_W, EMB_DIM), jnp.float32),
          pltpu.VMEM((B_PER_W, EMB_DIM), jnp.float32),
          pltpu.VMEM((B_PER_W,), jnp.float32),
          pltpu.SemaphoreType.DMA,
      ],
  )
  heads2 = heads.astype(jnp.int32).reshape(NUM_WORKERS * N_CHUNKS, CHUNK)
  rels2 = rels.astype(jnp.int32).reshape(NUM_WORKERS * N_CHUNKS, CHUNK)
  tails2 = tails.astype(jnp.int32).reshape(NUM_WORKERS * N_CHUNKS, CHUNK)
  return run(heads2, rels2, tails2, eh, et, rf, ri)


def kernel(heads, rels, tails, ent_embeds_head, ent_embeds_tail,
           rel_embeds_for, rel_embeds_inv):
  return _simple_sc(heads, rels, tails, ent_embeds_head, ent_embeds_tail,
                    rel_embeds_for, rel_embeds_inv)
